# baseline (speedup 1.0000x reference)
import functools

import jax
import jax.numpy as jnp
from jax import lax
from jax.experimental import pallas as pl
from jax.experimental.pallas import tpu as pltpu
from jax.experimental.pallas import tpu_sc as plsc

M = 16
K = 512
T = 512
NW = 32
KPW = K // NW

BEV_PX = 2048
CENTER = 256.0
MAX_SPEED = 15.0

NWORDS = BEV_PX * BEV_PX // 4
WPT = NWORDS // 16
QSTEP = 0.9 / 254.0

CH = KPW * T

BM = 4
BK = 128
MB = M // BM
KB = K // BK


def _sc_body(widx_hbm, tab_hbm, words_out, idxb, outb, spm_tab,
             sem_in, sem_out, sem_gat):
    sid = lax.axis_index("s")
    wid = lax.axis_index("c") * 16 + sid
    jbase = wid * KPW

    def in_dma(m, slot):
        return pltpu.make_async_copy(
            widx_hbm.at[pl.ds((m * K + jbase) * T, CH)],
            idxb.at[pl.ds(slot * CH, CH)], sem_in)

    def out_dma(m, slot):
        return pltpu.make_async_copy(
            outb.at[pl.ds(slot * CH, CH)],
            words_out.at[pl.ds((m * K + jbase) * T, CH)], sem_out)

    pltpu.sync_copy(tab_hbm.at[pl.ds(sid * WPT, WPT)],
                    spm_tab.at[pl.ds(sid * WPT, WPT)])
    in_dma(0, 0).start()
    plsc.subcore_barrier()

    def m_body(m, carry):
        slot = m & 1
        in_dma(m, slot).wait()
        pl.when(m + 1 < M)(lambda: in_dma(m + 1, 1 - slot).start())
        pl.when(m >= 2)(lambda: out_dma(m - 2, slot).wait())
        pltpu.async_copy(spm_tab.at[idxb.at[pl.ds(slot * CH, CH)]],
                         outb.at[pl.ds(slot * CH, CH)], sem_gat).wait()
        out_dma(m, slot).start()
        return carry

    lax.fori_loop(0, M, m_body, 0)
    out_dma(M - 2, 0).wait()
    out_dma(M - 1, 1).wait()


@functools.cache
def _sc_gather():
    return pl.kernel(
        _sc_body,
        out_type=jax.ShapeDtypeStruct((M * K * T,), jnp.int32),
        mesh=plsc.VectorSubcoreMesh(core_axis_name="c", subcore_axis_name="s"),
        compiler_params=pltpu.CompilerParams(needs_layout_passes=False),
        scratch_types=[
            pltpu.VMEM((2 * CH,), jnp.int32),
            pltpu.VMEM((2 * CH,), jnp.int32),
            pltpu.VMEM_SHARED((NWORDS,), jnp.int32),
            pltpu.SemaphoreType.DMA,
            pltpu.SemaphoreType.DMA,
            pltpu.SemaphoreType.DMA,
        ],
    )


def _tab_body(b_ref, o_ref):
    v = b_ref[...]
    s = v * v
    q = jnp.round(s * (1.0 / QSTEP))
    o_ref[...] = jnp.where(s >= 0.9, jnp.float32(255.0), q).astype(jnp.uint8)


def _make_table(BEVmap):
    tab = pl.pallas_call(
        _tab_body,
        grid=(8,),
        in_specs=[pl.BlockSpec((256, BEV_PX), lambda i: (i, 0))],
        out_specs=pl.BlockSpec((256, BEV_PX), lambda i: (i, 0)),
        out_shape=jax.ShapeDtypeStruct((BEV_PX, BEV_PX), jnp.uint8),
    )(BEVmap)
    return lax.bitcast_convert_type(tab.reshape(NWORDS, 4), jnp.int32)


def _pa_body(x_ref, y_ref, yaw_ref, vel_ref, g_ref,
             widx_ref, sh_ref, cm_ref, ctg_ref):
    mb = pl.program_id(1)
    x = x_ref[...]
    y = y_ref[...]
    ix = ((x + CENTER) * 0.25).astype(jnp.int32)
    iy = ((y + CENTER) * 0.25).astype(jnp.int32)
    e = (iy << 11) + ix
    widx_ref[...] = e >> 2
    sh_ref[...] = ((e & 3) << 3).astype(jnp.uint8)
    vel = vel_ref[...]
    yaw = yaw_ref[...]
    vc = jnp.sqrt(jnp.abs(MAX_SPEED - vel) * (1.0 / MAX_SPEED))
    ay = vel * yaw
    ac = ay * ay
    ac = jnp.where(ac > 25.0, jnp.float32(100.0), ac)
    cms = jnp.sum(1.5 * vc + 0.01 * ac, axis=0)
    g = g_ref[...]
    dx = x[:, :, T - 1] - g[0, 0]
    dy = y[:, :, T - 1] - g[0, 1]
    ct = jnp.sum(jnp.sqrt(dx * dx + dy * dy), axis=0).reshape(1, BK)

    @pl.when(mb == 0)
    def _():
        cm_ref[...] = cms
        ctg_ref[...] = ct

    @pl.when(mb > 0)
    def _():
        cm_ref[...] += cms
        ctg_ref[...] += ct

    @pl.when(mb == MB - 1)
    def _():
        ctg_ref[...] = ctg_ref[...] * (1.0 / M)


def _pass_a(x, y, yaw, vel, goal):
    comp_spec = pl.BlockSpec((BM, BK, T), lambda kb, mb: (mb, kb, 0))
    return pl.pallas_call(
        _pa_body,
        grid=(KB, MB),
        in_specs=[comp_spec, comp_spec, comp_spec, comp_spec,
                  pl.BlockSpec((1, 2), lambda kb, mb: (0, 0))],
        out_specs=[
            pl.BlockSpec((BM, BK, T), lambda kb, mb: (mb, kb, 0)),
            pl.BlockSpec((BM, BK, T), lambda kb, mb: (mb, kb, 0)),
            pl.BlockSpec((BK, T), lambda kb, mb: (kb, 0)),
            pl.BlockSpec((1, BK), lambda kb, mb: (0, kb)),
        ],
        out_shape=[
            jax.ShapeDtypeStruct((M, K, T), jnp.int32),
            jax.ShapeDtypeStruct((M, K, T), jnp.uint8),
            jax.ShapeDtypeStruct((K, T), jnp.float32),
            jax.ShapeDtypeStruct((1, K), jnp.float32),
        ],
    )(x, y, yaw, vel, goal.reshape(1, 2))


def _pb_body(w_ref, sh_ref, cm_ref, ctg_ref, o_ref):
    mb = pl.program_id(1)
    w = w_ref[...]
    sh = sh_ref[...].astype(jnp.int32)
    bits = (w >> sh) & 0xFF
    sc = jnp.where(bits == 255, jnp.float32(100.0),
                   bits.astype(jnp.float32) * QSTEP)
    scs = jnp.sum(sc, axis=0)

    @pl.when(mb == 0)
    def _():
        o_ref[...] = scs

    @pl.when(mb > 0)
    def _():
        o_ref[...] += scs

    @pl.when(mb == MB - 1)
    def _():
        o_ref[...] = (o_ref[...] + cm_ref[...]) * (1.0 / M) + ctg_ref[...]


def _pass_b(words, sh, cm, ctg):
    blk = pl.BlockSpec((BM, BK, T), lambda kb, mb: (mb, kb, 0))
    return pl.pallas_call(
        _pb_body,
        grid=(KB, MB),
        in_specs=[blk, blk, pl.BlockSpec((BK, T), lambda kb, mb: (kb, 0)),
                  pl.BlockSpec((1, T), lambda kb, mb: (0, 0))],
        out_specs=pl.BlockSpec((BK, T), lambda kb, mb: (kb, 0)),
        out_shape=jax.ShapeDtypeStruct((K, T), jnp.float32),
    )(words, sh, cm, ctg)


def kernel(state, BEVmap, goal_state):
    tab = _make_table(BEVmap)
    xt = jnp.moveaxis(state, -1, 0)
    widx, sh, cm, ctg = _pass_a(xt[0], xt[1], xt[2], xt[3],
                                goal_state.astype(jnp.float32))
    words = _sc_gather()(widx.reshape(-1), tab).reshape(M, K, T)
    return _pass_b(words, sh, cm, ctg)
